# Initial kernel scaffold; baseline (speedup 1.0000x reference)
import functools

import jax
import jax.numpy as jnp
from jax import lax
from jax.experimental import pallas as pl
from jax.experimental.pallas import tpu as pltpu
from jax.experimental.pallas import tpu_sc as plsc

DIM = 64
OUT_D = 2 * DIM
GRP = 128
CHUNK_GRPS = 5
CHUNK = GRP * CHUNK_GRPS
NC = 2
NS = 16
NW = NC * NS


@functools.partial(jax.jit, static_argnums=(4,))
def _embed(qtab, ctab, qidx, cidx, n_rows):
  rows_per_w = n_rows // NW
  grps_per_w = rows_per_w // GRP
  chunks_per_w = grps_per_w // CHUNK_GRPS
  mesh = plsc.VectorSubcoreMesh(core_axis_name="c", subcore_axis_name="s")

  @functools.partial(
      pl.kernel,
      out_type=jax.ShapeDtypeStruct((n_rows, OUT_D), jnp.float32),
      mesh=mesh,
      scratch_types=[
          pltpu.VMEM((grps_per_w, GRP), jnp.int32),
          pltpu.VMEM((grps_per_w, GRP), jnp.int32),
          pltpu.VMEM((CHUNK, OUT_D), jnp.float32),
          pltpu.SemaphoreType.DMA,
      ],
  )
  def k(qtab, ctab, qidx, cidx, out, qidx_v, cidx_v, buf, sem):
    wid = lax.axis_index("s") * NC + lax.axis_index("c")
    gbase = wid * grps_per_w
    rbase = wid * rows_per_w
    pltpu.sync_copy(qidx.at[pl.ds(gbase, grps_per_w)], qidx_v)
    pltpu.sync_copy(cidx.at[pl.ds(gbase, grps_per_w)], cidx_v)

    def chunk(ch, carry):
      g0 = ch * CHUNK_GRPS
      cps = []
      for j in range(CHUNK_GRPS):
        r0 = j * GRP
        cps.append(pltpu.make_async_copy(
            qtab.at[qidx_v.at[g0 + j]],
            buf.at[pl.ds(r0, GRP), pl.ds(0, DIM)], sem))
        cps.append(pltpu.make_async_copy(
            ctab.at[cidx_v.at[g0 + j]],
            buf.at[pl.ds(r0, GRP), pl.ds(DIM, DIM)], sem))
      for cp in cps:
        cp.start()
      for cp in cps:
        cp.wait()
      pltpu.sync_copy(buf, out.at[pl.ds(rbase + ch * CHUNK, CHUNK)])
      return carry

    lax.fori_loop(0, chunks_per_w, chunk, 0)

  return k(qtab, ctab, qidx, cidx)


def kernel(question_table, correctness_table, question_index, correctness_index):
  batch, hist = question_index.shape
  n_rows = batch * hist
  qidx = question_index.reshape(-1, GRP).astype(jnp.int32)
  cidx = correctness_index.reshape(-1, GRP).astype(jnp.int32)
  out = _embed(question_table, correctness_table, qidx, cidx, n_rows)
  return out.reshape(batch, hist, OUT_D)


# speedup vs baseline: 1.5601x; 1.5601x over previous
import functools

import jax
import jax.numpy as jnp
from jax import lax
from jax.experimental import pallas as pl
from jax.experimental.pallas import tpu as pltpu
from jax.experimental.pallas import tpu_sc as plsc

DIM = 64
OUT_D = 2 * DIM
GRP = 128
CHUNK_GRPS = 5
CHUNK = GRP * CHUNK_GRPS
LANE = 16
NC = 2
NS = 16
NW = NC * NS


@functools.partial(jax.jit, static_argnums=(5,))
def _embed(qtab2, ctab2, qhalf, qoff, coff, n_rows):
  rows_per_w = n_rows // NW
  grps_per_w = rows_per_w // GRP
  chunks_per_w = grps_per_w // CHUNK_GRPS
  mesh = plsc.VectorSubcoreMesh(core_axis_name="c", subcore_axis_name="s")

  @functools.partial(
      pl.kernel,
      out_type=jax.ShapeDtypeStruct((n_rows, OUT_D), jnp.float32),
      mesh=mesh,
      scratch_types=[
          pltpu.VMEM((grps_per_w, GRP), jnp.int32),
          pltpu.VMEM((grps_per_w, GRP), jnp.int32),
          pltpu.VMEM((grps_per_w, GRP), jnp.int32),
          pltpu.VMEM((1, OUT_D), jnp.float32),
          pltpu.VMEM((CHUNK, OUT_D), jnp.float32),
          pltpu.SemaphoreType.DMA,
      ],
  )
  def k(qtab2, ctab2, qhalf, qoff, coff, out, qh_v, qo_v, co_v, ct_v, buf,
        sem):
    wid = lax.axis_index("s") * NC + lax.axis_index("c")
    rbase = wid * rows_per_w
    pltpu.sync_copy(qhalf.at[wid], qh_v)
    pltpu.sync_copy(qoff.at[wid], qo_v)
    pltpu.sync_copy(coff.at[wid], co_v)
    pltpu.sync_copy(ctab2, ct_v)

    def chunk(ch, carry):
      g0 = ch * CHUNK_GRPS
      cps = []
      for j in range(CHUNK_GRPS):
        cps.append(pltpu.make_async_copy(
            qtab2.at[qh_v.at[g0 + j]], buf.at[pl.ds(j * GRP, GRP)], sem))
      for cp in cps:
        cp.start()
      for cp in cps:
        cp.wait()

      for g in range(CHUNK_GRPS):
        def fix(gl, c2):
          qo16 = qo_v[g0 + g, pl.ds(gl * LANE, LANE)]
          co16 = co_v[g0 + g, pl.ds(gl * LANE, LANE)]
          r0 = g * GRP + gl * LANE
          for i in range(LANE):
            r = r0 + i
            p_off = qo16[i]
            c_off = co16[i]
            for kk in range(DIM // LANE):
              buf[r, pl.ds(kk * LANE, LANE)] = (
                  buf[r, pl.ds(p_off + kk * LANE, LANE)])
            for kk in range(DIM // LANE):
              buf[r, pl.ds(DIM + kk * LANE, LANE)] = (
                  ct_v[0, pl.ds(c_off + kk * LANE, LANE)])
          return c2

        lax.fori_loop(0, GRP // LANE, fix, 0)

      pltpu.sync_copy(buf, out.at[pl.ds(rbase + ch * CHUNK, CHUNK)])
      return carry

    lax.fori_loop(0, chunks_per_w, chunk, 0)

  return k(qtab2, ctab2, qhalf, qoff, coff)


def kernel(question_table, correctness_table, question_index, correctness_index):
  batch, hist = question_index.shape
  n_rows = batch * hist
  qtab2 = question_table.reshape(-1, OUT_D)
  ctab2 = correctness_table.reshape(1, OUT_D)
  qi = question_index.reshape(NW, -1, GRP).astype(jnp.int32)
  ci = correctness_index.reshape(NW, -1, GRP).astype(jnp.int32)
  qhalf = qi >> 1
  qoff = (qi & 1) * DIM
  coff = ci * DIM
  out = _embed(qtab2, ctab2, qhalf, qoff, coff, n_rows)
  return out.reshape(batch, hist, OUT_D)


# speedup vs baseline: 2.0135x; 1.2906x over previous
import functools

import jax
import jax.numpy as jnp
from jax import lax
from jax.experimental import pallas as pl
from jax.experimental.pallas import tpu as pltpu
from jax.experimental.pallas import tpu_sc as plsc

DIM = 64
OUT_D = 2 * DIM
GRP = 128
CHUNK_GRPS = 1
CHUNK = GRP * CHUNK_GRPS
NBUF = 2
LANE = 16
NC = 2
NS = 16
NW = NC * NS


@functools.partial(jax.jit, static_argnums=(5,))
def _embed(qtab2, ctab2, qhalf, qoff, coff, n_rows):
  rows_per_w = n_rows // NW
  grps_per_w = rows_per_w // GRP
  chunks_per_w = grps_per_w // CHUNK_GRPS
  assert grps_per_w % CHUNK_GRPS == 0 and chunks_per_w % NBUF == 0
  mesh = plsc.VectorSubcoreMesh(core_axis_name="c", subcore_axis_name="s")

  @functools.partial(
      pl.kernel,
      out_type=jax.ShapeDtypeStruct((n_rows, OUT_D), jnp.float32),
      mesh=mesh,
      scratch_types=[
          pltpu.VMEM((grps_per_w, GRP), jnp.int32),
          pltpu.VMEM((grps_per_w, GRP), jnp.int32),
          pltpu.VMEM((grps_per_w, GRP), jnp.int32),
          pltpu.VMEM((1, OUT_D), jnp.float32),
          pltpu.VMEM((CHUNK, OUT_D), jnp.float32),
          pltpu.VMEM((CHUNK, OUT_D), jnp.float32),
          pltpu.SemaphoreType.DMA,
          pltpu.SemaphoreType.DMA,
      ],
  )
  def k(qtab2, ctab2, qhalf, qoff, coff, out, qh_v, qo_v, co_v, ct_v,
        buf0, buf1, sem0, sem1):
    wid = lax.axis_index("s") * NC + lax.axis_index("c")
    rbase = wid * rows_per_w
    pltpu.sync_copy(qhalf.at[wid], qh_v)
    pltpu.sync_copy(qoff.at[wid], qo_v)
    pltpu.sync_copy(coff.at[wid], co_v)
    pltpu.sync_copy(ctab2, ct_v)
    bufs = (buf0, buf1)
    sems = (sem0, sem1)

    def fire(ch, buf, sem):
      g0 = ch * CHUNK_GRPS
      for j in range(CHUNK_GRPS):
        pltpu.make_async_copy(
            qtab2.at[qh_v.at[g0 + j]], buf.at[pl.ds(j * GRP, GRP)],
            sem).start()

    def drain(buf, sem):
      for j in range(CHUNK_GRPS):
        pltpu.make_async_copy(
            qtab2.at[qh_v.at[0]], buf.at[pl.ds(j * GRP, GRP)], sem).wait()

    def fixup(ch, buf):
      g0 = ch * CHUNK_GRPS
      for g in range(CHUNK_GRPS):
        def fix(gl, c2):
          qo16 = qo_v[g0 + g, pl.ds(gl * LANE, LANE)]
          co16 = co_v[g0 + g, pl.ds(gl * LANE, LANE)]
          r0 = g * GRP + gl * LANE
          for i in range(LANE):
            r = r0 + i
            p_off = qo16[i]
            c_off = co16[i]
            for kk in range(DIM // LANE):
              buf[r, pl.ds(kk * LANE, LANE)] = (
                  buf[r, pl.ds(p_off + kk * LANE, LANE)])
            for kk in range(DIM // LANE):
              buf[r, pl.ds(DIM + kk * LANE, LANE)] = (
                  ct_v[0, pl.ds(c_off + kk * LANE, LANE)])
          return c2

        lax.fori_loop(0, GRP // LANE, fix, 0)

    for b in range(NBUF):
      fire(b, bufs[b], sems[b])

    def step(ph, carry):
      for b in range(NBUF):
        ch = ph * NBUF + b
        drain(bufs[b], sems[b])
        fixup(ch, bufs[b])
        pltpu.sync_copy(bufs[b], out.at[pl.ds(rbase + ch * CHUNK, CHUNK)])

        @pl.when(ch + NBUF < chunks_per_w)
        def _():
          fire(ch + NBUF, bufs[b], sems[b])
      return carry

    lax.fori_loop(0, chunks_per_w // NBUF, step, 0)

  return k(qtab2, ctab2, qhalf, qoff, coff)


def kernel(question_table, correctness_table, question_index, correctness_index):
  batch, hist = question_index.shape
  n_rows = batch * hist
  qtab2 = question_table.reshape(-1, OUT_D)
  ctab2 = correctness_table.reshape(1, OUT_D)
  qi = question_index.T.reshape(NW, -1, GRP).astype(jnp.int32)
  ci = correctness_index.T.reshape(NW, -1, GRP).astype(jnp.int32)
  qhalf = qi >> 1
  qoff = (qi & 1) * DIM
  coff = ci * DIM
  out = _embed(qtab2, ctab2, qhalf, qoff, coff, n_rows)
  return out.reshape(hist, batch, OUT_D).transpose(1, 0, 2)
